# Initial kernel scaffold; baseline (speedup 1.0000x reference)
import jax
import jax.numpy as jnp
from jax.experimental import pallas as pl


def kernel(node_features, lattices, frac_diff, W_e1, b_e1, W_e2, b_e2, W_n1, b_n1, W_n2, b_n2, ln_g, ln_b, edge_index, edge2graph, num_atoms):
    raise NotImplementedError("write your pallas kernel here")


# speedup vs baseline: 4.0683x; 4.0683x over previous
import functools

import jax
import jax.numpy as jnp
from jax import lax
from jax.experimental import pallas as pl
from jax.experimental.pallas import tpu as pltpu
from jax.experimental.pallas import tpu_sc as plsc

N = 10000
E = 320000
G = 16
H = 128

GW = 128
NWIN = E // GW
GWG = 64
NWING = E // GWG
EB = 2000
NEB = E // EB
NP = 10240
ROWS_PER_TILE = NP // 16


def _silu(v):
    return v * jax.nn.sigmoid(v)


def _prep_body(nf_ref, lng_ref, lnb_ref, ws_ref, wd_ref, lat_ref, wlat_ref,
               be1_ref, x_ref, xps_ref, xpd_ref, latb_ref):
    nf = nf_ref[...]
    mu = jnp.mean(nf, axis=1, keepdims=True)
    var = jnp.mean((nf - mu) ** 2, axis=1, keepdims=True)
    x = (nf - mu) * lax.rsqrt(var + 1e-5) * lng_ref[...] + lnb_ref[...]
    x_ref[...] = x
    xps_ref[...] = jnp.dot(x, ws_ref[...], preferred_element_type=jnp.float32)
    xpd_ref[...] = jnp.dot(x, wd_ref[...], preferred_element_type=jnp.float32)
    latb_ref[...] = (jnp.dot(lat_ref[...], wlat_ref[...],
                             preferred_element_type=jnp.float32) + be1_ref[...])


def _node_prep(nf, lng, lnb, ws, wd, lat8, wlat8, be1):
    return pl.pallas_call(
        _prep_body,
        out_shape=[
            jax.ShapeDtypeStruct((N, H), jnp.float32),
            jax.ShapeDtypeStruct((N, H), jnp.float32),
            jax.ShapeDtypeStruct((N, H), jnp.float32),
            jax.ShapeDtypeStruct((G, H), jnp.float32),
        ],
    )(nf, lng, lnb, ws, wd, lat8, wlat8, be1)


def _gather_one(table, idx2d):
    mesh = plsc.VectorSubcoreMesh(core_axis_name="c", subcore_axis_name="s")

    @functools.partial(
        pl.kernel,
        out_type=jax.ShapeDtypeStruct((E, H), jnp.float32),
        mesh=mesh,
    )
    def k(tab_hbm, idx_hbm, o_hbm):
        def body(i_vmem, o_vmem):
            pltpu.sync_copy(tab_hbm.at[i_vmem.at[0]], o_vmem)

        pltpu.emit_pipeline(
            body,
            grid=(NWIN,),
            in_specs=[pl.BlockSpec((1, GW), lambda i: (0, i))],
            out_specs=[pl.BlockSpec((GW, H), lambda i: (i, 0))],
            core_axis_name=("c", "s"),
            dimension_semantics=(pltpu.PARALLEL,),
        )(idx_hbm, o_hbm)

    return k(table, idx2d)


def _edge_body(hi_ref, hj_ref, fd_ref, s0_ref, s1_ref, latb_ref, wfd_ref,
               we2_ref, be2_ref, o_ref):
    i = pl.program_id(0)
    z = hi_ref[...] + hj_ref[...]
    z = z + jnp.dot(fd_ref[...], wfd_ref[...],
                    preferred_element_type=jnp.float32)
    row = lax.broadcasted_iota(jnp.int32, (EB, G), 0) + i * EB
    oh = jnp.logical_and(row >= s0_ref[...], row < s1_ref[...])
    z = z + jnp.dot(oh.astype(jnp.float32), latb_ref[...],
                    preferred_element_type=jnp.float32)
    a = _silu(z)
    b = jnp.dot(a, we2_ref[...], preferred_element_type=jnp.float32) + be2_ref[...]
    o_ref[...] = _silu(b)


def _edge_mlp(hi, hj, fd8, s0, s1, latb, wfd8, we2, be2):
    return pl.pallas_call(
        _edge_body,
        grid=(NEB,),
        in_specs=[
            pl.BlockSpec((EB, H), lambda i: (i, 0)),
            pl.BlockSpec((EB, H), lambda i: (i, 0)),
            pl.BlockSpec((EB, 8), lambda i: (i, 0)),
            pl.BlockSpec((1, G), lambda i: (0, 0)),
            pl.BlockSpec((1, G), lambda i: (0, 0)),
            pl.BlockSpec((G, H), lambda i: (0, 0)),
            pl.BlockSpec((8, H), lambda i: (0, 0)),
            pl.BlockSpec((H, H), lambda i: (0, 0)),
            pl.BlockSpec((1, H), lambda i: (0, 0)),
        ],
        out_specs=pl.BlockSpec((EB, H), lambda i: (i, 0)),
        out_shape=jax.ShapeDtypeStruct((E, H), jnp.float32),
    )(hi, hj, fd8, s0, s1, latb, wfd8, we2, be2)


def _scatter_stage(ef2, src2d, zacc):
    mesh = plsc.VectorSubcoreMesh(core_axis_name="c", subcore_axis_name="s")

    @functools.partial(
        pl.kernel,
        out_type=jax.ShapeDtypeStruct((2, NP, H), jnp.float32),
        mesh=mesh,
        scratch_types=[pltpu.VMEM_SHARED((NP, H), jnp.float32)],
    )
    def k(ef2_hbm, src_hbm, zacc_hbm, oacc_hbm, acc_sh):
        cid = lax.axis_index("c")
        sid = lax.axis_index("s")

        @pl.loop(0, ROWS_PER_TILE, step=128)
        def _(r):
            csl = pl.ds(sid * ROWS_PER_TILE + r, 128)
            pltpu.sync_copy(zacc_hbm.at[csl], acc_sh.at[csl])

        plsc.subcore_barrier()

        def body(x_vmem, i_vmem):
            pltpu.sync_copy(x_vmem, acc_sh.at[i_vmem.at[0]], add=True)

        pltpu.emit_pipeline(
            body,
            grid=(NWIN,),
            in_specs=[
                pl.BlockSpec((GW, H), lambda i: (i, 0)),
                pl.BlockSpec((1, GW), lambda i: (0, i)),
            ],
            out_specs=[],
            core_axis_name=("c", "s"),
            dimension_semantics=(pltpu.PARALLEL,),
        )(ef2_hbm, src_hbm)

        plsc.subcore_barrier()

        @pl.loop(0, ROWS_PER_TILE, step=128)
        def _(r):
            csl = pl.ds(sid * ROWS_PER_TILE + r, 128)
            pltpu.sync_copy(acc_sh.at[csl], oacc_hbm.at[cid, csl])

    return k(ef2, src2d, zacc)


def _count_stage(src2d, zcnt):
    mesh = plsc.VectorSubcoreMesh(core_axis_name="c", subcore_axis_name="s")

    @functools.partial(
        pl.kernel,
        out_type=jax.ShapeDtypeStruct((2, NP, H), jnp.float32),
        mesh=mesh,
        scratch_types=[
            pltpu.VMEM_SHARED((NP, H), jnp.float32),
            pltpu.VMEM((GW, H), jnp.float32),
        ],
    )
    def k(src_hbm, zcnt_hbm, ocnt_hbm, cnt_sh, ones_v):
        cid = lax.axis_index("c")
        sid = lax.axis_index("s")

        @pl.loop(0, ROWS_PER_TILE, step=128)
        def _(r):
            csl = pl.ds(sid * ROWS_PER_TILE + r, 128)
            pltpu.sync_copy(zcnt_hbm.at[csl], cnt_sh.at[csl])

        @pl.loop(0, GW)
        def _(i):
            for j in range(H // 16):
                ones_v[i, pl.ds(j * 16, 16)] = jnp.full((16,), 1.0, jnp.float32)

        plsc.subcore_barrier()

        def body(i_vmem):
            pltpu.sync_copy(ones_v, cnt_sh.at[i_vmem.at[0]], add=True)

        pltpu.emit_pipeline(
            body,
            grid=(NWIN,),
            in_specs=[pl.BlockSpec((1, GW), lambda i: (0, i))],
            out_specs=[],
            core_axis_name=("c", "s"),
            dimension_semantics=(pltpu.PARALLEL,),
        )(src_hbm)

        plsc.subcore_barrier()

        @pl.loop(0, ROWS_PER_TILE, step=128)
        def _(r):
            csl = pl.ds(sid * ROWS_PER_TILE + r, 128)
            pltpu.sync_copy(cnt_sh.at[csl], ocnt_hbm.at[cid, csl])

    return k(src2d, zcnt)


def _node_body(ni_ref, x_ref, part_ref, cnt_ref, w1a_ref, w1b_ref, b1_ref,
               w2_ref, b2_ref, o_ref):
    agg = part_ref[0:N, :] + part_ref[NP:NP + N, :]
    c = cnt_ref[0:N, 0:1] + cnt_ref[NP:NP + N, 0:1]
    mean = agg / jnp.maximum(c, 1.0)
    x = x_ref[...]
    h = _silu(jnp.dot(x, w1a_ref[...], preferred_element_type=jnp.float32)
              + jnp.dot(mean, w1b_ref[...], preferred_element_type=jnp.float32)
              + b1_ref[...])
    h = _silu(jnp.dot(h, w2_ref[...], preferred_element_type=jnp.float32)
              + b2_ref[...])
    o_ref[...] = ni_ref[...] + h


def _node_mlp(ni, x, part, cnt, w1a, w1b, b1, w2, b2):
    return pl.pallas_call(
        _node_body,
        out_shape=jax.ShapeDtypeStruct((N, H), jnp.float32),
    )(ni, x, part, cnt, w1a, w1b, b1, w2, b2)


def kernel(node_features, lattices, frac_diff, W_e1, b_e1, W_e2, b_e2,
           W_n1, b_n1, W_n2, b_n2, ln_g, ln_b, edge_index, edge2graph,
           num_atoms):
    del num_atoms
    src = edge_index[0]
    dst = edge_index[1]
    src2d = src.reshape(1, E)
    dst2d = dst.reshape(1, E)

    lat8 = jnp.concatenate(
        [lattices.reshape(G, 6), jnp.zeros((G, 2), jnp.float32)], axis=1)
    wlat8 = jnp.concatenate(
        [W_e1[2 * H:2 * H + 6], jnp.zeros((2, H), jnp.float32)], axis=0)
    fd8 = jnp.concatenate(
        [frac_diff, jnp.zeros((E, 5), jnp.float32)], axis=1)
    wfd8 = jnp.concatenate(
        [W_e1[2 * H + 6:], jnp.zeros((5, H), jnp.float32)], axis=0)

    x, xps, xpd, latb = _node_prep(
        node_features, ln_g.reshape(1, H), ln_b.reshape(1, H),
        W_e1[:H], W_e1[H:2 * H], lat8, wlat8, b_e1.reshape(1, H))

    hi_g = _gather_one(xps, src2d)
    hj_g = _gather_one(xpd, dst2d)

    starts = jnp.searchsorted(
        edge2graph, jnp.arange(G + 1, dtype=jnp.int32)).astype(jnp.int32)
    s0 = starts[:G].reshape(1, G)
    s1 = starts[1:].reshape(1, G)

    ef2 = _edge_mlp(hi_g, hj_g, fd8, s0, s1, latb, wfd8, W_e2,
                    b_e2.reshape(1, H))

    part = _scatter_stage(ef2, src2d, jnp.zeros((NP, H), jnp.float32))
    cnt = _count_stage(src2d, jnp.zeros((NP, H), jnp.float32))

    return _node_mlp(
        node_features, x, part.reshape(2 * NP, H), cnt.reshape(2 * NP, H),
        W_n1[:H], W_n1[H:], b_n1.reshape(1, H), W_n2, b_n2.reshape(1, H))


# speedup vs baseline: 4.1723x; 1.0256x over previous
import functools

import jax
import jax.numpy as jnp
from jax import lax
from jax.experimental import pallas as pl
from jax.experimental.pallas import tpu as pltpu
from jax.experimental.pallas import tpu_sc as plsc

N = 10000
E = 320000
G = 16
H = 128

GW = 128
EB = 2000
NP = 10240
ROWS_PER_TILE = NP // 16
EHALF = E // 2


def _silu(v):
    return v * jax.nn.sigmoid(v)


def _prep_body(nf_ref, lng_ref, lnb_ref, ws_ref, wd_ref, lat_ref, wlat_ref,
               be1_ref, x_ref, xps_ref, xpd_ref, latb_ref):
    nf = nf_ref[...]
    mu = jnp.mean(nf, axis=1, keepdims=True)
    var = jnp.mean((nf - mu) ** 2, axis=1, keepdims=True)
    x = (nf - mu) * lax.rsqrt(var + 1e-5) * lng_ref[...] + lnb_ref[...]
    x_ref[...] = x
    xps_ref[...] = jnp.dot(x, ws_ref[...], preferred_element_type=jnp.float32)
    xpd_ref[...] = jnp.dot(x, wd_ref[...], preferred_element_type=jnp.float32)
    latb_ref[...] = (jnp.dot(lat_ref[...], wlat_ref[...],
                             preferred_element_type=jnp.float32) + be1_ref[...])


def _node_prep(nf, lng, lnb, ws, wd, lat8, wlat8, be1):
    return pl.pallas_call(
        _prep_body,
        out_shape=[
            jax.ShapeDtypeStruct((N, H), jnp.float32),
            jax.ShapeDtypeStruct((N, H), jnp.float32),
            jax.ShapeDtypeStruct((N, H), jnp.float32),
            jax.ShapeDtypeStruct((G, H), jnp.float32),
        ],
    )(nf, lng, lnb, ws, wd, lat8, wlat8, be1)


def _gather_one(table, idx2d):
    ne = idx2d.shape[1]
    mesh = plsc.VectorSubcoreMesh(core_axis_name="c", subcore_axis_name="s")

    @functools.partial(
        pl.kernel,
        out_type=jax.ShapeDtypeStruct((ne, H), jnp.float32),
        mesh=mesh,
    )
    def k(tab_hbm, idx_hbm, o_hbm):
        def body(i_vmem, o_vmem):
            pltpu.sync_copy(tab_hbm.at[i_vmem.at[0]], o_vmem)

        pltpu.emit_pipeline(
            body,
            grid=(ne // GW,),
            in_specs=[pl.BlockSpec((1, GW), lambda i: (0, i))],
            out_specs=[pl.BlockSpec((GW, H), lambda i: (i, 0))],
            core_axis_name=("c", "s"),
            dimension_semantics=(pltpu.PARALLEL,),
        )(idx_hbm, o_hbm)

    return k(table, idx2d)


def _edge_mlp(hi, hj, fd8, s0, s1, latb, wfd8, we2, be2, ebase):
    ne = hi.shape[0]

    def body(hi_ref, hj_ref, fd_ref, s0_ref, s1_ref, latb_ref, wfd_ref,
             we2_ref, be2_ref, o_ref):
        i = pl.program_id(0)
        z = hi_ref[...] + hj_ref[...]
        z = z + jnp.dot(fd_ref[...], wfd_ref[...],
                        preferred_element_type=jnp.float32)
        row = lax.broadcasted_iota(jnp.int32, (EB, G), 0) + (i * EB + ebase)
        oh = jnp.logical_and(row >= s0_ref[...], row < s1_ref[...])
        z = z + jnp.dot(oh.astype(jnp.float32), latb_ref[...],
                        preferred_element_type=jnp.float32)
        a = _silu(z)
        b = (jnp.dot(a, we2_ref[...], preferred_element_type=jnp.float32)
             + be2_ref[...])
        o_ref[...] = _silu(b)

    return pl.pallas_call(
        body,
        grid=(ne // EB,),
        in_specs=[
            pl.BlockSpec((EB, H), lambda i: (i, 0)),
            pl.BlockSpec((EB, H), lambda i: (i, 0)),
            pl.BlockSpec((EB, 8), lambda i: (i, 0)),
            pl.BlockSpec((1, G), lambda i: (0, 0)),
            pl.BlockSpec((1, G), lambda i: (0, 0)),
            pl.BlockSpec((G, H), lambda i: (0, 0)),
            pl.BlockSpec((8, H), lambda i: (0, 0)),
            pl.BlockSpec((H, H), lambda i: (0, 0)),
            pl.BlockSpec((1, H), lambda i: (0, 0)),
        ],
        out_specs=pl.BlockSpec((EB, H), lambda i: (i, 0)),
        out_shape=jax.ShapeDtypeStruct((ne, H), jnp.float32),
    )(hi, hj, fd8, s0, s1, latb, wfd8, we2, be2)


def _scatter_stage(ef2, src2d, zacc):
    ne = src2d.shape[1]
    mesh = plsc.VectorSubcoreMesh(core_axis_name="c", subcore_axis_name="s")

    @functools.partial(
        pl.kernel,
        out_type=jax.ShapeDtypeStruct((2, NP, H), jnp.float32),
        mesh=mesh,
        scratch_types=[pltpu.VMEM_SHARED((NP, H), jnp.float32)],
    )
    def k(ef2_hbm, src_hbm, zacc_hbm, oacc_hbm, acc_sh):
        cid = lax.axis_index("c")
        sid = lax.axis_index("s")

        @pl.loop(0, ROWS_PER_TILE, step=128)
        def _(r):
            csl = pl.ds(sid * ROWS_PER_TILE + r, 128)
            pltpu.sync_copy(zacc_hbm.at[csl], acc_sh.at[csl])

        plsc.subcore_barrier()

        def body(x_vmem, i_vmem):
            pltpu.sync_copy(x_vmem, acc_sh.at[i_vmem.at[0]], add=True)

        pltpu.emit_pipeline(
            body,
            grid=(ne // GW,),
            in_specs=[
                pl.BlockSpec((GW, H), lambda i: (i, 0)),
                pl.BlockSpec((1, GW), lambda i: (0, i)),
            ],
            out_specs=[],
            core_axis_name=("c", "s"),
            dimension_semantics=(pltpu.PARALLEL,),
        )(ef2_hbm, src_hbm)

        plsc.subcore_barrier()

        @pl.loop(0, ROWS_PER_TILE, step=128)
        def _(r):
            csl = pl.ds(sid * ROWS_PER_TILE + r, 128)
            pltpu.sync_copy(acc_sh.at[csl], oacc_hbm.at[cid, csl])

    return k(ef2, src2d, zacc)


def _count_stage(src2d, zcnt):
    ne = src2d.shape[1]
    mesh = plsc.VectorSubcoreMesh(core_axis_name="c", subcore_axis_name="s")

    @functools.partial(
        pl.kernel,
        out_type=jax.ShapeDtypeStruct((2, NP, H), jnp.float32),
        mesh=mesh,
        scratch_types=[
            pltpu.VMEM_SHARED((NP, H), jnp.float32),
            pltpu.VMEM((GW, H), jnp.float32),
        ],
    )
    def k(src_hbm, zcnt_hbm, ocnt_hbm, cnt_sh, ones_v):
        cid = lax.axis_index("c")
        sid = lax.axis_index("s")

        @pl.loop(0, ROWS_PER_TILE, step=128)
        def _(r):
            csl = pl.ds(sid * ROWS_PER_TILE + r, 128)
            pltpu.sync_copy(zcnt_hbm.at[csl], cnt_sh.at[csl])

        @pl.loop(0, GW)
        def _(i):
            for j in range(H // 16):
                ones_v[i, pl.ds(j * 16, 16)] = jnp.full((16,), 1.0, jnp.float32)

        plsc.subcore_barrier()

        def body(i_vmem):
            pltpu.sync_copy(ones_v, cnt_sh.at[i_vmem.at[0]], add=True)

        pltpu.emit_pipeline(
            body,
            grid=(ne // GW,),
            in_specs=[pl.BlockSpec((1, GW), lambda i: (0, i))],
            out_specs=[],
            core_axis_name=("c", "s"),
            dimension_semantics=(pltpu.PARALLEL,),
        )(src_hbm)

        plsc.subcore_barrier()

        @pl.loop(0, ROWS_PER_TILE, step=128)
        def _(r):
            csl = pl.ds(sid * ROWS_PER_TILE + r, 128)
            pltpu.sync_copy(cnt_sh.at[csl], ocnt_hbm.at[cid, csl])

    return k(src2d, zcnt)


def _node_body(ni_ref, x_ref, pa_ref, pb_ref, cnt_ref, w1a_ref, w1b_ref,
               b1_ref, w2_ref, b2_ref, o_ref):
    agg = (pa_ref[0:N, :] + pa_ref[NP:NP + N, :]
           + pb_ref[0:N, :] + pb_ref[NP:NP + N, :])
    c = cnt_ref[0:N, 0:1] + cnt_ref[NP:NP + N, 0:1]
    mean = agg / jnp.maximum(c, 1.0)
    x = x_ref[...]
    h = _silu(jnp.dot(x, w1a_ref[...], preferred_element_type=jnp.float32)
              + jnp.dot(mean, w1b_ref[...], preferred_element_type=jnp.float32)
              + b1_ref[...])
    h = _silu(jnp.dot(h, w2_ref[...], preferred_element_type=jnp.float32)
              + b2_ref[...])
    o_ref[...] = ni_ref[...] + h


def _node_mlp(ni, x, part_a, part_b, cnt, w1a, w1b, b1, w2, b2):
    return pl.pallas_call(
        _node_body,
        out_shape=jax.ShapeDtypeStruct((N, H), jnp.float32),
    )(ni, x, part_a, part_b, cnt, w1a, w1b, b1, w2, b2)


def kernel(node_features, lattices, frac_diff, W_e1, b_e1, W_e2, b_e2,
           W_n1, b_n1, W_n2, b_n2, ln_g, ln_b, edge_index, edge2graph,
           num_atoms):
    del num_atoms
    src = edge_index[0].reshape(1, E)
    dst = edge_index[1].reshape(1, E)
    src_a, src_b = src[:, :EHALF], src[:, EHALF:]
    dst_a, dst_b = dst[:, :EHALF], dst[:, EHALF:]

    lat8 = jnp.concatenate(
        [lattices.reshape(G, 6), jnp.zeros((G, 2), jnp.float32)], axis=1)
    wlat8 = jnp.concatenate(
        [W_e1[2 * H:2 * H + 6], jnp.zeros((2, H), jnp.float32)], axis=0)
    fd8 = jnp.concatenate(
        [frac_diff, jnp.zeros((E, 5), jnp.float32)], axis=1)
    wfd8 = jnp.concatenate(
        [W_e1[2 * H + 6:], jnp.zeros((5, H), jnp.float32)], axis=0)

    x, xps, xpd, latb = _node_prep(
        node_features, ln_g.reshape(1, H), ln_b.reshape(1, H),
        W_e1[:H], W_e1[H:2 * H], lat8, wlat8, b_e1.reshape(1, H))

    starts = jnp.searchsorted(
        edge2graph, jnp.arange(G + 1, dtype=jnp.int32)).astype(jnp.int32)
    s0 = starts[:G].reshape(1, G)
    s1 = starts[1:].reshape(1, G)
    be2 = b_e2.reshape(1, H)
    zacc = jnp.zeros((NP, H), jnp.float32)

    hi_a = _gather_one(xps, src_a)
    hj_a = _gather_one(xpd, dst_a)
    ef2_a = _edge_mlp(hi_a, hj_a, fd8[:EHALF], s0, s1, latb, wfd8, W_e2,
                      be2, 0)

    hi_b = _gather_one(xps, src_b)
    hj_b = _gather_one(xpd, dst_b)
    ef2_b = _edge_mlp(hi_b, hj_b, fd8[EHALF:], s0, s1, latb, wfd8, W_e2,
                      be2, EHALF)

    cnt = _count_stage(src, zacc)
    part_a = _scatter_stage(ef2_a, src_a, zacc)
    part_b = _scatter_stage(ef2_b, src_b, zacc)

    return _node_mlp(
        node_features, x, part_a.reshape(2 * NP, H), part_b.reshape(2 * NP, H),
        cnt.reshape(2 * NP, H),
        W_n1[:H], W_n1[H:], b_n1.reshape(1, H), W_n2, b_n2.reshape(1, H))


# speedup vs baseline: 4.7851x; 1.1469x over previous
import functools

import jax
import jax.numpy as jnp
from jax import lax
from jax.experimental import pallas as pl
from jax.experimental.pallas import tpu as pltpu
from jax.experimental.pallas import tpu_sc as plsc

N = 10000
E = 320000
G = 16
H = 128

GW = 128
EB = 2000
NP = 10240
ROWS_PER_TILE = NP // 16
EHALF = E // 2


def _silu(v):
    return v * jax.nn.sigmoid(v)


def _prep_body(nf_ref, lng_ref, lnb_ref, ws_ref, wd_ref, lat_ref, wlat_ref,
               be1_ref, e2g_ref, x_ref, xps_ref, xpd_ref, latb_ref,
               s0_ref, s1_ref):
    nf = nf_ref[...]
    mu = jnp.mean(nf, axis=1, keepdims=True)
    var = jnp.mean((nf - mu) ** 2, axis=1, keepdims=True)
    x = (nf - mu) * lax.rsqrt(var + 1e-5) * lng_ref[...] + lnb_ref[...]
    x_ref[...] = x
    xps_ref[...] = jnp.dot(x, ws_ref[...], preferred_element_type=jnp.float32)
    xpd_ref[...] = jnp.dot(x, wd_ref[...], preferred_element_type=jnp.float32)
    latb_ref[...] = (jnp.dot(lat_ref[...], wlat_ref[...],
                             preferred_element_type=jnp.float32) + be1_ref[...])
    e2g = e2g_ref[...]
    lane = lax.broadcasted_iota(jnp.int32, (1, G), 1)
    c_le = jnp.zeros((1, G), jnp.int32)
    for g in range(G):
        cnt = jnp.sum((e2g <= g).astype(jnp.int32))
        c_le = c_le + jnp.where(lane == g, cnt, 0)
    s1_ref[...] = c_le
    s0_ref[...] = jnp.concatenate(
        [jnp.zeros((1, 1), jnp.int32), c_le[:, :G - 1]], axis=1)


def _node_prep(nf, lng, lnb, ws, wd, lat8, wlat8, be1, e2g_r):
    return pl.pallas_call(
        _prep_body,
        out_shape=[
            jax.ShapeDtypeStruct((N, H), jnp.float32),
            jax.ShapeDtypeStruct((N, H), jnp.float32),
            jax.ShapeDtypeStruct((N, H), jnp.float32),
            jax.ShapeDtypeStruct((G, H), jnp.float32),
            jax.ShapeDtypeStruct((1, G), jnp.int32),
            jax.ShapeDtypeStruct((1, G), jnp.int32),
        ],
    )(nf, lng, lnb, ws, wd, lat8, wlat8, be1, e2g_r)


def _gather_one(table, idx2d):
    ne = idx2d.shape[1]
    mesh = plsc.VectorSubcoreMesh(core_axis_name="c", subcore_axis_name="s")

    @functools.partial(
        pl.kernel,
        out_type=jax.ShapeDtypeStruct((ne, H), jnp.float32),
        mesh=mesh,
    )
    def k(tab_hbm, idx_hbm, o_hbm):
        def body(i_vmem, o_vmem):
            pltpu.sync_copy(tab_hbm.at[i_vmem.at[0]], o_vmem)

        pltpu.emit_pipeline(
            body,
            grid=(ne // GW,),
            in_specs=[pl.BlockSpec((1, GW), lambda i: (0, i))],
            out_specs=[pl.BlockSpec((GW, H), lambda i: (i, 0))],
            core_axis_name=("c", "s"),
            dimension_semantics=(pltpu.PARALLEL,),
        )(idx_hbm, o_hbm)

    return k(table, idx2d)


def _edge_mlp(hi, hj, fd8, s0, s1, latb, wfd8, we2, be2, ebase):
    ne = hi.shape[0]

    def body(hi_ref, hj_ref, fd_ref, s0_ref, s1_ref, latb_ref, wfd_ref,
             we2_ref, be2_ref, o_ref):
        i = pl.program_id(0)
        z = hi_ref[...] + hj_ref[...]
        z = (z + fd_ref[:, 0:1] * wfd_ref[0:1, :]
             + fd_ref[:, 1:2] * wfd_ref[1:2, :]
             + fd_ref[:, 2:3] * wfd_ref[2:3, :])
        row = lax.broadcasted_iota(jnp.int32, (EB, G), 0) + (i * EB + ebase)
        oh = jnp.logical_and(row >= s0_ref[...], row < s1_ref[...])
        z = z + jnp.dot(oh.astype(jnp.float32), latb_ref[...],
                        preferred_element_type=jnp.float32)
        a = _silu(z)
        b = (jnp.dot(a, we2_ref[...], preferred_element_type=jnp.float32)
             + be2_ref[...])
        o_ref[...] = _silu(b)

    return pl.pallas_call(
        body,
        grid=(ne // EB,),
        in_specs=[
            pl.BlockSpec((EB, H), lambda i: (i, 0)),
            pl.BlockSpec((EB, H), lambda i: (i, 0)),
            pl.BlockSpec((EB, 3), lambda i: (i, 0)),
            pl.BlockSpec((1, G), lambda i: (0, 0)),
            pl.BlockSpec((1, G), lambda i: (0, 0)),
            pl.BlockSpec((G, H), lambda i: (0, 0)),
            pl.BlockSpec((3, H), lambda i: (0, 0)),
            pl.BlockSpec((H, H), lambda i: (0, 0)),
            pl.BlockSpec((1, H), lambda i: (0, 0)),
        ],
        out_specs=pl.BlockSpec((EB, H), lambda i: (i, 0)),
        out_shape=jax.ShapeDtypeStruct((ne, H), jnp.float32),
    )(hi, hj, fd8, s0, s1, latb, wfd8, we2, be2)


def _scatter_stage(ef2, src2d, zacc):
    ne = src2d.shape[1]
    mesh = plsc.VectorSubcoreMesh(core_axis_name="c", subcore_axis_name="s")

    @functools.partial(
        pl.kernel,
        out_type=jax.ShapeDtypeStruct((2, NP, H), jnp.float32),
        mesh=mesh,
        scratch_types=[pltpu.VMEM_SHARED((NP, H), jnp.float32)],
    )
    def k(ef2_hbm, src_hbm, zacc_hbm, oacc_hbm, acc_sh):
        cid = lax.axis_index("c")
        sid = lax.axis_index("s")

        @pl.loop(0, ROWS_PER_TILE, step=128)
        def _(r):
            csl = pl.ds(sid * ROWS_PER_TILE + r, 128)
            pltpu.sync_copy(zacc_hbm.at[csl], acc_sh.at[csl])

        plsc.subcore_barrier()

        def body(x_vmem, i_vmem):
            pltpu.sync_copy(x_vmem, acc_sh.at[i_vmem.at[0]], add=True)

        pltpu.emit_pipeline(
            body,
            grid=(ne // GW,),
            in_specs=[
                pl.BlockSpec((GW, H), lambda i: (i, 0)),
                pl.BlockSpec((1, GW), lambda i: (0, i)),
            ],
            out_specs=[],
            core_axis_name=("c", "s"),
            dimension_semantics=(pltpu.PARALLEL,),
        )(ef2_hbm, src_hbm)

        plsc.subcore_barrier()

        @pl.loop(0, ROWS_PER_TILE, step=128)
        def _(r):
            csl = pl.ds(sid * ROWS_PER_TILE + r, 128)
            pltpu.sync_copy(acc_sh.at[csl], oacc_hbm.at[cid, csl])

    return k(ef2, src2d, zacc)


def _count_stage(src2d, zcnt):
    ne = src2d.shape[1]
    mesh = plsc.VectorSubcoreMesh(core_axis_name="c", subcore_axis_name="s")

    @functools.partial(
        pl.kernel,
        out_type=jax.ShapeDtypeStruct((2, NP, H), jnp.float32),
        mesh=mesh,
        scratch_types=[
            pltpu.VMEM_SHARED((NP, H), jnp.float32),
            pltpu.VMEM((GW, H), jnp.float32),
        ],
    )
    def k(src_hbm, zcnt_hbm, ocnt_hbm, cnt_sh, ones_v):
        cid = lax.axis_index("c")
        sid = lax.axis_index("s")

        @pl.loop(0, ROWS_PER_TILE, step=128)
        def _(r):
            csl = pl.ds(sid * ROWS_PER_TILE + r, 128)
            pltpu.sync_copy(zcnt_hbm.at[csl], cnt_sh.at[csl])

        @pl.loop(0, GW)
        def _(i):
            for j in range(H // 16):
                ones_v[i, pl.ds(j * 16, 16)] = jnp.full((16,), 1.0, jnp.float32)

        plsc.subcore_barrier()

        def body(i_vmem):
            pltpu.sync_copy(ones_v, cnt_sh.at[i_vmem.at[0]], add=True)

        pltpu.emit_pipeline(
            body,
            grid=(ne // GW,),
            in_specs=[pl.BlockSpec((1, GW), lambda i: (0, i))],
            out_specs=[],
            core_axis_name=("c", "s"),
            dimension_semantics=(pltpu.PARALLEL,),
        )(src_hbm)

        plsc.subcore_barrier()

        @pl.loop(0, ROWS_PER_TILE, step=128)
        def _(r):
            csl = pl.ds(sid * ROWS_PER_TILE + r, 128)
            pltpu.sync_copy(cnt_sh.at[csl], ocnt_hbm.at[cid, csl])

    return k(src2d, zcnt)


def _node_body(ni_ref, x_ref, pa_ref, pb_ref, cnt_ref, w1a_ref, w1b_ref,
               b1_ref, w2_ref, b2_ref, o_ref):
    agg = (pa_ref[0:N, :] + pa_ref[NP:NP + N, :]
           + pb_ref[0:N, :] + pb_ref[NP:NP + N, :])
    c = cnt_ref[0:N, 0:1] + cnt_ref[NP:NP + N, 0:1]
    mean = agg / jnp.maximum(c, 1.0)
    x = x_ref[...]
    h = _silu(jnp.dot(x, w1a_ref[...], preferred_element_type=jnp.float32)
              + jnp.dot(mean, w1b_ref[...], preferred_element_type=jnp.float32)
              + b1_ref[...])
    h = _silu(jnp.dot(h, w2_ref[...], preferred_element_type=jnp.float32)
              + b2_ref[...])
    o_ref[...] = ni_ref[...] + h


def _node_mlp(ni, x, part_a, part_b, cnt, w1a, w1b, b1, w2, b2):
    return pl.pallas_call(
        _node_body,
        out_shape=jax.ShapeDtypeStruct((N, H), jnp.float32),
    )(ni, x, part_a, part_b, cnt, w1a, w1b, b1, w2, b2)


def kernel(node_features, lattices, frac_diff, W_e1, b_e1, W_e2, b_e2,
           W_n1, b_n1, W_n2, b_n2, ln_g, ln_b, edge_index, edge2graph,
           num_atoms):
    del num_atoms
    src = edge_index[0].reshape(1, E)
    dst = edge_index[1].reshape(1, E)
    src_a, src_b = src[:, :EHALF], src[:, EHALF:]
    dst_a, dst_b = dst[:, :EHALF], dst[:, EHALF:]

    lat8 = jnp.concatenate(
        [lattices.reshape(G, 6), jnp.zeros((G, 2), jnp.float32)], axis=1)
    wlat8 = jnp.concatenate(
        [W_e1[2 * H:2 * H + 6], jnp.zeros((2, H), jnp.float32)], axis=0)
    wfd = W_e1[2 * H + 6:]

    x, xps, xpd, latb, s0, s1 = _node_prep(
        node_features, ln_g.reshape(1, H), ln_b.reshape(1, H),
        W_e1[:H], W_e1[H:2 * H], lat8, wlat8, b_e1.reshape(1, H),
        edge2graph.reshape(E // GW, GW))

    be2 = b_e2.reshape(1, H)
    zacc = jnp.zeros((NP, H), jnp.float32)

    hi_a = _gather_one(xps, src_a)
    hj_a = _gather_one(xpd, dst_a)
    ef2_a = _edge_mlp(hi_a, hj_a, frac_diff[:EHALF], s0, s1, latb, wfd, W_e2,
                      be2, 0)

    hi_b = _gather_one(xps, src_b)
    hj_b = _gather_one(xpd, dst_b)
    ef2_b = _edge_mlp(hi_b, hj_b, frac_diff[EHALF:], s0, s1, latb, wfd, W_e2,
                      be2, EHALF)

    cnt = _count_stage(src, zacc)
    part_a = _scatter_stage(ef2_a, src_a, zacc)
    part_b = _scatter_stage(ef2_b, src_b, zacc)

    return _node_mlp(
        node_features, x, part_a.reshape(2 * NP, H), part_b.reshape(2 * NP, H),
        cnt.reshape(2 * NP, H),
        W_n1[:H], W_n1[H:], b_n1.reshape(1, H), W_n2, b_n2.reshape(1, H))


# speedup vs baseline: 5.3323x; 1.1144x over previous
import functools

import jax
import jax.numpy as jnp
from jax import lax
from jax.experimental import pallas as pl
from jax.experimental.pallas import tpu as pltpu
from jax.experimental.pallas import tpu_sc as plsc

N = 10000
E = 320000
G = 16
H = 128

GW = 128
EB = 3200
NP = 10240
ROWS_PER_TILE = NP // 16
EHALF = E // 2


def _silu(v):
    return v * jax.nn.sigmoid(v)


def _prep_body(nf_ref, lng_ref, lnb_ref, ws_ref, wd_ref, lat_ref, wlat_ref,
               be1_ref, e2g_ref, x_ref, xps_ref, xpd_ref, latb_ref,
               s0_ref, s1_ref):
    nf = nf_ref[...]
    mu = jnp.mean(nf, axis=1, keepdims=True)
    var = jnp.mean((nf - mu) ** 2, axis=1, keepdims=True)
    x = (nf - mu) * lax.rsqrt(var + 1e-5) * lng_ref[...] + lnb_ref[...]
    x_ref[...] = x
    xps_ref[...] = jnp.dot(x, ws_ref[...], preferred_element_type=jnp.float32)
    xpd_ref[...] = jnp.dot(x, wd_ref[...], preferred_element_type=jnp.float32)
    latb_ref[...] = (jnp.dot(lat_ref[...], wlat_ref[...],
                             preferred_element_type=jnp.float32) + be1_ref[...])
    e2g = e2g_ref[...]
    lane = lax.broadcasted_iota(jnp.int32, (1, G), 1)
    c_le = jnp.zeros((1, G), jnp.int32)
    for g in range(G):
        cnt = jnp.sum((e2g <= g).astype(jnp.int32))
        c_le = c_le + jnp.where(lane == g, cnt, 0)
    s1_ref[...] = c_le
    s0_ref[...] = jnp.concatenate(
        [jnp.zeros((1, 1), jnp.int32), c_le[:, :G - 1]], axis=1)


def _node_prep(nf, lng, lnb, ws, wd, lat8, wlat8, be1, e2g_r):
    return pl.pallas_call(
        _prep_body,
        out_shape=[
            jax.ShapeDtypeStruct((N, H), jnp.float32),
            jax.ShapeDtypeStruct((N, H), jnp.float32),
            jax.ShapeDtypeStruct((N, H), jnp.float32),
            jax.ShapeDtypeStruct((G, H), jnp.float32),
            jax.ShapeDtypeStruct((1, G), jnp.int32),
            jax.ShapeDtypeStruct((1, G), jnp.int32),
        ],
    )(nf, lng, lnb, ws, wd, lat8, wlat8, be1, e2g_r)


def _gather_one(table, idx2d):
    ne = idx2d.shape[1]
    mesh = plsc.VectorSubcoreMesh(core_axis_name="c", subcore_axis_name="s")

    @functools.partial(
        pl.kernel,
        out_type=jax.ShapeDtypeStruct((ne, H), jnp.float32),
        mesh=mesh,
    )
    def k(tab_hbm, idx_hbm, o_hbm):
        def body(i_vmem, o_vmem):
            pltpu.sync_copy(tab_hbm.at[i_vmem.at[0]], o_vmem)

        pltpu.emit_pipeline(
            body,
            grid=(ne // GW,),
            in_specs=[pl.BlockSpec((1, GW), lambda i: (0, i))],
            out_specs=[pl.BlockSpec((GW, H), lambda i: (i, 0))],
            core_axis_name=("c", "s"),
            dimension_semantics=(pltpu.PARALLEL,),
        )(idx_hbm, o_hbm)

    return k(table, idx2d)


def _edge_mlp(hi, hj, fd8, s0, s1, latb, wfd8, we2, be2, ebase):
    ne = hi.shape[0]

    def body(hi_ref, hj_ref, fd_ref, s0_ref, s1_ref, latb_ref, wfd_ref,
             we2_ref, be2_ref, o_ref):
        i = pl.program_id(0)
        z = hi_ref[...] + hj_ref[...]
        z = z + lax.dot_general(fd_ref[...], wfd_ref[...],
                                (((0,), (0,)), ((), ())),
                                preferred_element_type=jnp.float32)
        row = lax.broadcasted_iota(jnp.int32, (EB, G), 0) + (i * EB + ebase)
        oh = jnp.logical_and(row >= s0_ref[...], row < s1_ref[...])
        z = z + jnp.dot(oh.astype(jnp.float32), latb_ref[...],
                        preferred_element_type=jnp.float32)
        a = _silu(z)
        b = (jnp.dot(a, we2_ref[...], preferred_element_type=jnp.float32)
             + be2_ref[...])
        o_ref[...] = _silu(b)

    return pl.pallas_call(
        body,
        grid=(ne // EB,),
        in_specs=[
            pl.BlockSpec((EB, H), lambda i: (i, 0)),
            pl.BlockSpec((EB, H), lambda i: (i, 0)),
            pl.BlockSpec((3, EB), lambda i: (0, i)),
            pl.BlockSpec((1, G), lambda i: (0, 0)),
            pl.BlockSpec((1, G), lambda i: (0, 0)),
            pl.BlockSpec((G, H), lambda i: (0, 0)),
            pl.BlockSpec((3, H), lambda i: (0, 0)),
            pl.BlockSpec((H, H), lambda i: (0, 0)),
            pl.BlockSpec((1, H), lambda i: (0, 0)),
        ],
        out_specs=pl.BlockSpec((EB, H), lambda i: (i, 0)),
        out_shape=jax.ShapeDtypeStruct((ne, H), jnp.float32),
    )(hi, hj, fd8, s0, s1, latb, wfd8, we2, be2)


def _scatter_stage(ef2, src2d, zacc):
    ne = src2d.shape[1]
    mesh = plsc.VectorSubcoreMesh(core_axis_name="c", subcore_axis_name="s")

    @functools.partial(
        pl.kernel,
        out_type=jax.ShapeDtypeStruct((2, NP, H), jnp.float32),
        mesh=mesh,
        scratch_types=[pltpu.VMEM_SHARED((NP, H), jnp.float32)],
    )
    def k(ef2_hbm, src_hbm, zacc_hbm, oacc_hbm, acc_sh):
        cid = lax.axis_index("c")
        sid = lax.axis_index("s")

        @pl.loop(0, ROWS_PER_TILE, step=128)
        def _(r):
            csl = pl.ds(sid * ROWS_PER_TILE + r, 128)
            pltpu.sync_copy(zacc_hbm.at[csl], acc_sh.at[csl])

        plsc.subcore_barrier()

        def body(x_vmem, i_vmem):
            pltpu.sync_copy(x_vmem, acc_sh.at[i_vmem.at[0]], add=True)

        pltpu.emit_pipeline(
            body,
            grid=(ne // GW,),
            in_specs=[
                pl.BlockSpec((GW, H), lambda i: (i, 0)),
                pl.BlockSpec((1, GW), lambda i: (0, i)),
            ],
            out_specs=[],
            core_axis_name=("c", "s"),
            dimension_semantics=(pltpu.PARALLEL,),
        )(ef2_hbm, src_hbm)

        plsc.subcore_barrier()

        @pl.loop(0, ROWS_PER_TILE, step=128)
        def _(r):
            csl = pl.ds(sid * ROWS_PER_TILE + r, 128)
            pltpu.sync_copy(acc_sh.at[csl], oacc_hbm.at[cid, csl])

    return k(ef2, src2d, zacc)


def _count_stage(src2d, zcnt):
    ne = src2d.shape[1]
    mesh = plsc.VectorSubcoreMesh(core_axis_name="c", subcore_axis_name="s")

    @functools.partial(
        pl.kernel,
        out_type=jax.ShapeDtypeStruct((2, NP, H), jnp.float32),
        mesh=mesh,
        scratch_types=[
            pltpu.VMEM_SHARED((NP, H), jnp.float32),
            pltpu.VMEM((GW, H), jnp.float32),
        ],
    )
    def k(src_hbm, zcnt_hbm, ocnt_hbm, cnt_sh, ones_v):
        cid = lax.axis_index("c")
        sid = lax.axis_index("s")

        @pl.loop(0, ROWS_PER_TILE, step=128)
        def _(r):
            csl = pl.ds(sid * ROWS_PER_TILE + r, 128)
            pltpu.sync_copy(zcnt_hbm.at[csl], cnt_sh.at[csl])

        @pl.loop(0, GW)
        def _(i):
            for j in range(H // 16):
                ones_v[i, pl.ds(j * 16, 16)] = jnp.full((16,), 1.0, jnp.float32)

        plsc.subcore_barrier()

        def body(i_vmem):
            pltpu.sync_copy(ones_v, cnt_sh.at[i_vmem.at[0]], add=True)

        pltpu.emit_pipeline(
            body,
            grid=(ne // GW,),
            in_specs=[pl.BlockSpec((1, GW), lambda i: (0, i))],
            out_specs=[],
            core_axis_name=("c", "s"),
            dimension_semantics=(pltpu.PARALLEL,),
        )(src_hbm)

        plsc.subcore_barrier()

        @pl.loop(0, ROWS_PER_TILE, step=128)
        def _(r):
            csl = pl.ds(sid * ROWS_PER_TILE + r, 128)
            pltpu.sync_copy(cnt_sh.at[csl], ocnt_hbm.at[cid, csl])

    return k(src2d, zcnt)


def _node_body(ni_ref, x_ref, pa_ref, pb_ref, cnt_ref, w1a_ref, w1b_ref,
               b1_ref, w2_ref, b2_ref, o_ref):
    agg = (pa_ref[0:N, :] + pa_ref[NP:NP + N, :]
           + pb_ref[0:N, :] + pb_ref[NP:NP + N, :])
    c = cnt_ref[0:N, 0:1] + cnt_ref[NP:NP + N, 0:1]
    mean = agg / jnp.maximum(c, 1.0)
    x = x_ref[...]
    h = _silu(jnp.dot(x, w1a_ref[...], preferred_element_type=jnp.float32)
              + jnp.dot(mean, w1b_ref[...], preferred_element_type=jnp.float32)
              + b1_ref[...])
    h = _silu(jnp.dot(h, w2_ref[...], preferred_element_type=jnp.float32)
              + b2_ref[...])
    o_ref[...] = ni_ref[...] + h


def _node_mlp(ni, x, part_a, part_b, cnt, w1a, w1b, b1, w2, b2):
    return pl.pallas_call(
        _node_body,
        out_shape=jax.ShapeDtypeStruct((N, H), jnp.float32),
    )(ni, x, part_a, part_b, cnt, w1a, w1b, b1, w2, b2)


def kernel(node_features, lattices, frac_diff, W_e1, b_e1, W_e2, b_e2,
           W_n1, b_n1, W_n2, b_n2, ln_g, ln_b, edge_index, edge2graph,
           num_atoms):
    del num_atoms
    src = edge_index[0].reshape(1, E)
    dst = edge_index[1].reshape(1, E)
    src_a, src_b = src[:, :EHALF], src[:, EHALF:]
    dst_a, dst_b = dst[:, :EHALF], dst[:, EHALF:]

    lat8 = jnp.concatenate(
        [lattices.reshape(G, 6), jnp.zeros((G, 2), jnp.float32)], axis=1)
    wlat8 = jnp.concatenate(
        [W_e1[2 * H:2 * H + 6], jnp.zeros((2, H), jnp.float32)], axis=0)
    wfd = W_e1[2 * H + 6:]
    fd_t = frac_diff.T

    x, xps, xpd, latb, s0, s1 = _node_prep(
        node_features, ln_g.reshape(1, H), ln_b.reshape(1, H),
        W_e1[:H], W_e1[H:2 * H], lat8, wlat8, b_e1.reshape(1, H),
        edge2graph.reshape(E // GW, GW))

    be2 = b_e2.reshape(1, H)
    zacc = jnp.zeros((NP, H), jnp.float32)

    hi_a = _gather_one(xps, src_a)
    hj_a = _gather_one(xpd, dst_a)
    ef2_a = _edge_mlp(hi_a, hj_a, fd_t[:, :EHALF], s0, s1, latb, wfd, W_e2,
                      be2, 0)

    hi_b = _gather_one(xps, src_b)
    hj_b = _gather_one(xpd, dst_b)
    ef2_b = _edge_mlp(hi_b, hj_b, fd_t[:, EHALF:], s0, s1, latb, wfd, W_e2,
                      be2, EHALF)

    cnt = _count_stage(src, zacc)
    part_a = _scatter_stage(ef2_a, src_a, zacc)
    part_b = _scatter_stage(ef2_b, src_b, zacc)

    return _node_mlp(
        node_features, x, part_a.reshape(2 * NP, H), part_b.reshape(2 * NP, H),
        cnt.reshape(2 * NP, H),
        W_n1[:H], W_n1[H:], b_n1.reshape(1, H), W_n2, b_n2.reshape(1, H))


# speedup vs baseline: 7.7188x; 1.4476x over previous
import functools

import jax
import jax.numpy as jnp
from jax import lax
from jax.experimental import pallas as pl
from jax.experimental.pallas import tpu as pltpu
from jax.experimental.pallas import tpu_sc as plsc

N = 10000
E = 320000
G = 16
H = 128

GW = 128
EB = 3200
NP = 10240
ROWS_PER_TILE = NP // 16
EHALF = E // 2


def _silu(v):
    return v * jax.nn.sigmoid(v)


def _prep_body(nf_ref, lng_ref, lnb_ref, ws_ref, wd_ref, lat_ref, wlat_ref,
               be1_ref, e2g_ref, x_ref, xps_ref, xpd_ref, latb_ref,
               s0_ref, s1_ref):
    nf = nf_ref[...]
    mu = jnp.mean(nf, axis=1, keepdims=True)
    var = jnp.mean((nf - mu) ** 2, axis=1, keepdims=True)
    x = (nf - mu) * lax.rsqrt(var + 1e-5) * lng_ref[...] + lnb_ref[...]
    x_ref[...] = x
    xps_ref[0:N, :] = jnp.dot(x, ws_ref[...],
                              preferred_element_type=jnp.float32)
    xpd_ref[0:N, :] = jnp.dot(x, wd_ref[...],
                              preferred_element_type=jnp.float32)
    latb_ref[...] = (jnp.dot(lat_ref[...], wlat_ref[...],
                             preferred_element_type=jnp.float32) + be1_ref[...])
    e2g = e2g_ref[...]
    lane = lax.broadcasted_iota(jnp.int32, (1, G), 1)
    c_le = jnp.zeros((1, G), jnp.int32)
    for g in range(G):
        cnt = jnp.sum((e2g <= g).astype(jnp.int32))
        c_le = c_le + jnp.where(lane == g, cnt, 0)
    s1_ref[...] = c_le
    s0_ref[...] = jnp.concatenate(
        [jnp.zeros((1, 1), jnp.int32), c_le[:, :G - 1]], axis=1)


def _node_prep(nf, lng, lnb, ws, wd, lat8, wlat8, be1, e2g_r):
    return pl.pallas_call(
        _prep_body,
        out_shape=[
            jax.ShapeDtypeStruct((N, H), jnp.float32),
            jax.ShapeDtypeStruct((NP, H), jnp.float32),
            jax.ShapeDtypeStruct((NP, H), jnp.float32),
            jax.ShapeDtypeStruct((G, H), jnp.float32),
            jax.ShapeDtypeStruct((1, G), jnp.int32),
            jax.ShapeDtypeStruct((1, G), jnp.int32),
        ],
    )(nf, lng, lnb, ws, wd, lat8, wlat8, be1, e2g_r)


def _gather_pair(xps_pad, xpd_pad, src2d, dst2d):
    ne = src2d.shape[1]
    mesh = plsc.VectorSubcoreMesh(core_axis_name="c", subcore_axis_name="s")

    @functools.partial(
        pl.kernel,
        out_type=(jax.ShapeDtypeStruct((ne, H), jnp.float32),
                  jax.ShapeDtypeStruct((ne, H), jnp.float32)),
        mesh=mesh,
        scratch_types=[pltpu.VMEM_SHARED((NP, H), jnp.float32)],
    )
    def k(xps_hbm, xpd_hbm, src_hbm, dst_hbm, ohi_hbm, ohj_hbm, tab_sh):
        cid = lax.axis_index("c")
        sid = lax.axis_index("s")

        @pl.loop(0, ROWS_PER_TILE, step=128)
        def _(r):
            csl = pl.ds(sid * ROWS_PER_TILE + r, 128)

            @pl.when(cid == 0)
            def _():
                pltpu.sync_copy(xps_hbm.at[csl], tab_sh.at[csl])

            @pl.when(cid == 1)
            def _():
                pltpu.sync_copy(xpd_hbm.at[csl], tab_sh.at[csl])

        plsc.subcore_barrier()

        def body(i_vmem, o_vmem):
            pltpu.sync_copy(tab_sh.at[i_vmem.at[0]], o_vmem)

        @pl.when(cid == 0)
        def _():
            pltpu.emit_pipeline(
                body,
                grid=(ne // GW,),
                in_specs=[pl.BlockSpec((1, GW), lambda i: (0, i))],
                out_specs=[pl.BlockSpec((GW, H), lambda i: (i, 0))],
                core_axis_name="s",
                dimension_semantics=(pltpu.PARALLEL,),
            )(src_hbm, ohi_hbm)

        @pl.when(cid == 1)
        def _():
            pltpu.emit_pipeline(
                body,
                grid=(ne // GW,),
                in_specs=[pl.BlockSpec((1, GW), lambda i: (0, i))],
                out_specs=[pl.BlockSpec((GW, H), lambda i: (i, 0))],
                core_axis_name="s",
                dimension_semantics=(pltpu.PARALLEL,),
            )(dst_hbm, ohj_hbm)

    return k(xps_pad, xpd_pad, src2d, dst2d)


def _edge_mlp(hi, hj, fd8, s0, s1, latb, wfd8, we2, be2, ebase):
    ne = hi.shape[0]

    def body(hi_ref, hj_ref, fd_ref, s0_ref, s1_ref, latb_ref, wfd_ref,
             we2_ref, be2_ref, o_ref):
        i = pl.program_id(0)
        z = hi_ref[...] + hj_ref[...]
        z = z + lax.dot_general(fd_ref[...], wfd_ref[...],
                                (((0,), (0,)), ((), ())),
                                preferred_element_type=jnp.float32)
        row = lax.broadcasted_iota(jnp.int32, (EB, G), 0) + (i * EB + ebase)
        oh = jnp.logical_and(row >= s0_ref[...], row < s1_ref[...])
        z = z + jnp.dot(oh.astype(jnp.float32), latb_ref[...],
                        preferred_element_type=jnp.float32)
        a = _silu(z)
        b = (jnp.dot(a, we2_ref[...], preferred_element_type=jnp.float32)
             + be2_ref[...])
        o_ref[...] = _silu(b)

    return pl.pallas_call(
        body,
        grid=(ne // EB,),
        in_specs=[
            pl.BlockSpec((EB, H), lambda i: (i, 0)),
            pl.BlockSpec((EB, H), lambda i: (i, 0)),
            pl.BlockSpec((3, EB), lambda i: (0, i)),
            pl.BlockSpec((1, G), lambda i: (0, 0)),
            pl.BlockSpec((1, G), lambda i: (0, 0)),
            pl.BlockSpec((G, H), lambda i: (0, 0)),
            pl.BlockSpec((3, H), lambda i: (0, 0)),
            pl.BlockSpec((H, H), lambda i: (0, 0)),
            pl.BlockSpec((1, H), lambda i: (0, 0)),
        ],
        out_specs=pl.BlockSpec((EB, H), lambda i: (i, 0)),
        out_shape=jax.ShapeDtypeStruct((ne, H), jnp.float32),
    )(hi, hj, fd8, s0, s1, latb, wfd8, we2, be2)


def _scatter_stage(ef2, src2d, zacc):
    ne = src2d.shape[1]
    mesh = plsc.VectorSubcoreMesh(core_axis_name="c", subcore_axis_name="s")

    @functools.partial(
        pl.kernel,
        out_type=jax.ShapeDtypeStruct((2, NP, H), jnp.float32),
        mesh=mesh,
        scratch_types=[pltpu.VMEM_SHARED((NP, H), jnp.float32)],
    )
    def k(ef2_hbm, src_hbm, zacc_hbm, oacc_hbm, acc_sh):
        cid = lax.axis_index("c")
        sid = lax.axis_index("s")

        @pl.loop(0, ROWS_PER_TILE, step=128)
        def _(r):
            csl = pl.ds(sid * ROWS_PER_TILE + r, 128)
            pltpu.sync_copy(zacc_hbm.at[csl], acc_sh.at[csl])

        plsc.subcore_barrier()

        def body(x_vmem, i_vmem):
            pltpu.sync_copy(x_vmem, acc_sh.at[i_vmem.at[0]], add=True)

        pltpu.emit_pipeline(
            body,
            grid=(ne // GW,),
            in_specs=[
                pl.BlockSpec((GW, H), lambda i: (i, 0)),
                pl.BlockSpec((1, GW), lambda i: (0, i)),
            ],
            out_specs=[],
            core_axis_name=("c", "s"),
            dimension_semantics=(pltpu.PARALLEL,),
        )(ef2_hbm, src_hbm)

        plsc.subcore_barrier()

        @pl.loop(0, ROWS_PER_TILE, step=128)
        def _(r):
            csl = pl.ds(sid * ROWS_PER_TILE + r, 128)
            pltpu.sync_copy(acc_sh.at[csl], oacc_hbm.at[cid, csl])

    return k(ef2, src2d, zacc)


def _count_stage(src2d, zcnt):
    ne = src2d.shape[1]
    mesh = plsc.VectorSubcoreMesh(core_axis_name="c", subcore_axis_name="s")

    @functools.partial(
        pl.kernel,
        out_type=jax.ShapeDtypeStruct((2, NP, H), jnp.float32),
        mesh=mesh,
        scratch_types=[
            pltpu.VMEM_SHARED((NP, H), jnp.float32),
            pltpu.VMEM((GW, H), jnp.float32),
        ],
    )
    def k(src_hbm, zcnt_hbm, ocnt_hbm, cnt_sh, ones_v):
        cid = lax.axis_index("c")
        sid = lax.axis_index("s")

        @pl.loop(0, ROWS_PER_TILE, step=128)
        def _(r):
            csl = pl.ds(sid * ROWS_PER_TILE + r, 128)
            pltpu.sync_copy(zcnt_hbm.at[csl], cnt_sh.at[csl])

        @pl.loop(0, GW)
        def _(i):
            for j in range(H // 16):
                ones_v[i, pl.ds(j * 16, 16)] = jnp.full((16,), 1.0, jnp.float32)

        plsc.subcore_barrier()

        def body(i_vmem):
            pltpu.sync_copy(ones_v, cnt_sh.at[i_vmem.at[0]], add=True)

        pltpu.emit_pipeline(
            body,
            grid=(ne // GW,),
            in_specs=[pl.BlockSpec((1, GW), lambda i: (0, i))],
            out_specs=[],
            core_axis_name=("c", "s"),
            dimension_semantics=(pltpu.PARALLEL,),
        )(src_hbm)

        plsc.subcore_barrier()

        @pl.loop(0, ROWS_PER_TILE, step=128)
        def _(r):
            csl = pl.ds(sid * ROWS_PER_TILE + r, 128)
            pltpu.sync_copy(cnt_sh.at[csl], ocnt_hbm.at[cid, csl])

    return k(src2d, zcnt)


def _node_body(ni_ref, x_ref, pa_ref, pb_ref, cnt_ref, w1a_ref, w1b_ref,
               b1_ref, w2_ref, b2_ref, o_ref):
    agg = (pa_ref[0:N, :] + pa_ref[NP:NP + N, :]
           + pb_ref[0:N, :] + pb_ref[NP:NP + N, :])
    c = cnt_ref[0:N, 0:1] + cnt_ref[NP:NP + N, 0:1]
    mean = agg / jnp.maximum(c, 1.0)
    x = x_ref[...]
    h = _silu(jnp.dot(x, w1a_ref[...], preferred_element_type=jnp.float32)
              + jnp.dot(mean, w1b_ref[...], preferred_element_type=jnp.float32)
              + b1_ref[...])
    h = _silu(jnp.dot(h, w2_ref[...], preferred_element_type=jnp.float32)
              + b2_ref[...])
    o_ref[...] = ni_ref[...] + h


def _node_mlp(ni, x, part_a, part_b, cnt, w1a, w1b, b1, w2, b2):
    return pl.pallas_call(
        _node_body,
        out_shape=jax.ShapeDtypeStruct((N, H), jnp.float32),
    )(ni, x, part_a, part_b, cnt, w1a, w1b, b1, w2, b2)


def kernel(node_features, lattices, frac_diff, W_e1, b_e1, W_e2, b_e2,
           W_n1, b_n1, W_n2, b_n2, ln_g, ln_b, edge_index, edge2graph,
           num_atoms):
    del num_atoms
    src = edge_index[0].reshape(1, E)
    dst = edge_index[1].reshape(1, E)
    src_a, src_b = src[:, :EHALF], src[:, EHALF:]
    dst_a, dst_b = dst[:, :EHALF], dst[:, EHALF:]

    lat8 = jnp.concatenate(
        [lattices.reshape(G, 6), jnp.zeros((G, 2), jnp.float32)], axis=1)
    wlat8 = jnp.concatenate(
        [W_e1[2 * H:2 * H + 6], jnp.zeros((2, H), jnp.float32)], axis=0)
    wfd = W_e1[2 * H + 6:]
    fd_t = frac_diff.T

    x, xps, xpd, latb, s0, s1 = _node_prep(
        node_features, ln_g.reshape(1, H), ln_b.reshape(1, H),
        W_e1[:H], W_e1[H:2 * H], lat8, wlat8, b_e1.reshape(1, H),
        edge2graph.reshape(E // GW, GW))

    be2 = b_e2.reshape(1, H)
    zacc = jnp.zeros((NP, H), jnp.float32)

    hi_a, hj_a = _gather_pair(xps, xpd, src_a, dst_a)
    ef2_a = _edge_mlp(hi_a, hj_a, fd_t[:, :EHALF], s0, s1, latb, wfd, W_e2,
                      be2, 0)

    hi_b, hj_b = _gather_pair(xps, xpd, src_b, dst_b)
    ef2_b = _edge_mlp(hi_b, hj_b, fd_t[:, EHALF:], s0, s1, latb, wfd, W_e2,
                      be2, EHALF)

    cnt = _count_stage(src, zacc)
    part_a = _scatter_stage(ef2_a, src_a, zacc)
    part_b = _scatter_stage(ef2_b, src_b, zacc)

    return _node_mlp(
        node_features, x, part_a.reshape(2 * NP, H), part_b.reshape(2 * NP, H),
        cnt.reshape(2 * NP, H),
        W_n1[:H], W_n1[H:], b_n1.reshape(1, H), W_n2, b_n2.reshape(1, H))
